# Initial kernel scaffold; baseline (speedup 1.0000x reference)
import jax
import jax.numpy as jnp
from jax.experimental import pallas as pl


def kernel(inputs, text_table, pos_table):
    raise NotImplementedError("write your pallas kernel here")


# speedup vs baseline: 6.2296x; 6.2296x over previous
import functools

import jax
import jax.numpy as jnp
from jax import lax
from jax.experimental import pallas as pl
from jax.experimental.pallas import tpu as pltpu
from jax.experimental.pallas import tpu_sc as plsc

_L = 200
_E = 128
_NW = 32
_HALF = _L // 2


def _build(batch):
  total = batch * _L
  chunks_per_w = total // (_NW * _L)
  assert chunks_per_w * _NW * _L == total

  mesh = plsc.VectorSubcoreMesh(core_axis_name="c", subcore_axis_name="s")

  @functools.partial(
      pl.kernel,
      mesh=mesh,
      out_type=jax.ShapeDtypeStruct((total, _E), jnp.float32),
      scratch_types=[
          pltpu.VMEM((2, 2, _HALF), jnp.int32),
          pltpu.VMEM((2, _L, _E), jnp.float32),
          pltpu.VMEM((_L, _E), jnp.float32),
          pltpu.SemaphoreType.DMA,
          pltpu.SemaphoreType.DMA,
      ],
  )
  def k(idx_hbm, text_hbm, pos_hbm, out_hbm, idx_v, buf_v, pos_v, g0, g1):
    wid = lax.axis_index("s") * 2 + lax.axis_index("c")
    base = wid * chunks_per_w
    gsem = (g0, g1)

    pltpu.sync_copy(pos_hbm, pos_v)

    def start_chunk(cl, slot):
      g = base + cl
      pltpu.sync_copy(idx_hbm.at[pl.ds(2 * g, 2)], idx_v.at[slot])
      pltpu.async_copy(text_hbm.at[idx_v.at[slot, 0]],
                       buf_v.at[slot, pl.ds(0, _HALF)], gsem[slot])
      pltpu.async_copy(text_hbm.at[idx_v.at[slot, 1]],
                       buf_v.at[slot, pl.ds(_HALF, _HALF)], gsem[slot])

    def finish_chunk(cl, slot):
      pltpu.make_async_copy(text_hbm.at[idx_v.at[slot, 0]],
                            buf_v.at[slot, pl.ds(0, _HALF)], gsem[slot]).wait()
      pltpu.make_async_copy(text_hbm.at[idx_v.at[slot, 1]],
                            buf_v.at[slot, pl.ds(_HALF, _HALF)],
                            gsem[slot]).wait()

      @pl.loop(0, _L, unroll=2)
      def _(r):
        for j in range(_E // 16):
          sl = pl.ds(j * 16, 16)
          plsc.addupdate(buf_v.at[slot, r, sl], pos_v[r, sl])

      g = base + cl
      pltpu.sync_copy(buf_v.at[slot], out_hbm.at[pl.ds(g * _L, _L)])

    start_chunk(0, 0)

    @pl.loop(0, chunks_per_w, step=2)
    def _(c0):
      for b in range(2):
        cl = c0 + b

        @pl.when(cl + 1 < chunks_per_w)
        def _():
          start_chunk(cl + 1, 1 - b)

        finish_chunk(cl, b)

  return k


def kernel(inputs, text_table, pos_table):
  batch, seq = inputs.shape
  assert seq == _L and text_table.shape[1] == _E
  idx2d = inputs.reshape(batch * _L // _HALF, _HALF).astype(jnp.int32)
  out = _build(batch)(idx2d, text_table, pos_table)
  return out.reshape(batch, _L, _E)
